# baseline (speedup 1.0000x reference)
import jax
import jax.numpy as jnp
from jax import lax
from jax.experimental import pallas as pl
from jax.experimental.pallas import tpu as pltpu
from jax.experimental.pallas import tpu_sc as plsc

_N = 131072
_ZROWS = 86
_D = 256
_L = 16

_NC = 2
_NS = 16
_NW = _NC * _NS
_BPW = _N // _NW
_C = 128
_NCHUNK = _BPW // _C
_TABW = _ZROWS * _D


def _prep_body(ec_ref, w_ref, zt_ref, tab_ref):
    t = (jnp.dot(ec_ref[...], w_ref[...], preferred_element_type=jnp.float32)
         + zt_ref[...])
    tab_ref[...] = jnp.broadcast_to(t[None], (_NW, _ZROWS, _D))


def _prep(ec86, w, zt):
    return pl.pallas_call(
        _prep_body,
        out_shape=jax.ShapeDtypeStruct((_NW, _ZROWS, _D), jnp.float32),
    )(ec86, w, zt)


def _gather_body(table_hbm, idx_hbm, out_hbm,
                 table_v, idx_v, bufs, osems):
    cid = lax.axis_index("c")
    sid = lax.axis_index("s")
    wid = sid * _NC + cid
    base = wid * _BPW

    pltpu.sync_copy(table_hbm.at[pl.ds(wid * _TABW, _TABW)], table_v)
    pltpu.sync_copy(idx_hbm.at[pl.ds(base, _BPW)], idx_v)

    iota16 = jax.lax.iota(jnp.int32, _L)

    def assemble(ci, buf):
        @pl.loop(0, _C // _L)
        def _grps(grp):
            zv = idx_v[pl.ds(ci * _C + grp * _L, _L)]
            src0 = zv * _D
            dstv = iota16 * _D + grp * (_L * _D)
            for c in range(_D):
                val = plsc.load_gather(table_v, [src0 + c])
                plsc.store_scatter(buf, [dstv + c], val)

    def start_scatter(ci, b):
        pltpu.async_copy(bufs[b],
                         out_hbm.at[pl.ds((base + ci * _C) * _D, _C * _D)],
                         osems[b])

    def wait_scatter(b):
        pltpu.make_async_copy(bufs[b],
                              out_hbm.at[pl.ds(base * _D, _C * _D)],
                              osems[b]).wait()

    @pl.loop(0, _NCHUNK, step=2)
    def _chunks(g):
        for b in range(2):
            ci = g + b

            @pl.when(ci >= 2)
            def _():
                wait_scatter(b)
            assemble(ci, bufs[b])
            start_scatter(ci, b)

    wait_scatter(0)
    wait_scatter(1)


def kernel(z, elec_config, m_weight, z_table):
    zi = z.astype(jnp.int32)
    tab = _prep(elec_config[:_ZROWS], m_weight, z_table).reshape(_NW * _TABW)
    mesh = plsc.VectorSubcoreMesh(core_axis_name="c", subcore_axis_name="s",
                                  num_cores=_NC, num_subcores=_NS)
    gather = pl.kernel(
        _gather_body,
        out_type=jax.ShapeDtypeStruct((_N * _D,), jnp.float32),
        mesh=mesh,
        compiler_params=pltpu.CompilerParams(needs_layout_passes=False),
        scratch_types=[
            pltpu.VMEM((_TABW,), jnp.float32),
            pltpu.VMEM((_BPW,), jnp.int32),
            [pltpu.VMEM((_C * _D,), jnp.float32) for _ in range(2)],
            [pltpu.SemaphoreType.DMA for _ in range(2)],
        ],
    )
    return gather(tab, zi).reshape(_N, _D)


# speedup vs baseline: 3.2160x; 3.2160x over previous
import jax
import jax.numpy as jnp
from jax import lax
from jax.experimental import pallas as pl
from jax.experimental.pallas import tpu as pltpu
from jax.experimental.pallas import tpu_sc as plsc

_N = 131072
_ZROWS = 86
_D = 256
_L = 16

_NC = 2
_NS = 16
_NW = _NC * _NS
_BPW = _N // _NW
_C = 128
_NCHUNK = _BPW // _C
_TABW = _ZROWS * _D


def _prep_body(ec_ref, w_ref, zt_ref, tab_ref):
    t = (jnp.dot(ec_ref[...], w_ref[...], preferred_element_type=jnp.float32)
         + zt_ref[...])
    tab_ref[...] = jnp.broadcast_to(t[None], (_NW, _ZROWS, _D))


def _prep(ec86, w, zt):
    return pl.pallas_call(
        _prep_body,
        out_shape=jax.ShapeDtypeStruct((_NW, _ZROWS, _D), jnp.float32),
    )(ec86, w, zt)


def _gather_body(table_hbm, idx_hbm, out_hbm,
                 table_v, idx_v, bufs, osems):
    cid = lax.axis_index("c")
    sid = lax.axis_index("s")
    wid = sid * _NC + cid
    base = wid * _BPW

    pltpu.sync_copy(table_hbm.at[pl.ds(wid * _TABW, _TABW)], table_v)
    pltpu.sync_copy(idx_hbm.at[pl.ds(base, _BPW)], idx_v)

    iota16 = jax.lax.iota(jnp.int32, _L)

    def assemble(ci, buf):
        @pl.loop(0, _C, unroll=8)
        def _rows(rr):
            ridx = ci * _C + rr
            zsplat = plsc.load_gather(
                idx_v, [jnp.full((_L,), ridx, jnp.int32)])
            zbase = zsplat * _D + iota16
            dst0 = rr * _D
            for c in range(_D // _L):
                val = plsc.load_gather(table_v, [zbase + (_L * c)])
                buf[pl.ds(dst0 + _L * c, _L)] = val

    def start_scatter(ci, b):
        pltpu.async_copy(bufs[b],
                         out_hbm.at[pl.ds((base + ci * _C) * _D, _C * _D)],
                         osems[b])

    def wait_scatter(b):
        pltpu.make_async_copy(bufs[b],
                              out_hbm.at[pl.ds(base * _D, _C * _D)],
                              osems[b]).wait()

    @pl.loop(0, _NCHUNK, step=2)
    def _chunks(g):
        for b in range(2):
            ci = g + b

            @pl.when(ci >= 2)
            def _():
                wait_scatter(b)
            assemble(ci, bufs[b])
            start_scatter(ci, b)

    wait_scatter(0)
    wait_scatter(1)


def kernel(z, elec_config, m_weight, z_table):
    zi = z.astype(jnp.int32)
    tab = _prep(elec_config[:_ZROWS], m_weight, z_table).reshape(_NW * _TABW)
    mesh = plsc.VectorSubcoreMesh(core_axis_name="c", subcore_axis_name="s",
                                  num_cores=_NC, num_subcores=_NS)
    gather = pl.kernel(
        _gather_body,
        out_type=jax.ShapeDtypeStruct((_N * _D,), jnp.float32),
        mesh=mesh,
        compiler_params=pltpu.CompilerParams(needs_layout_passes=False),
        scratch_types=[
            pltpu.VMEM((_TABW,), jnp.float32),
            pltpu.VMEM((_BPW,), jnp.int32),
            [pltpu.VMEM((_C * _D,), jnp.float32) for _ in range(2)],
            [pltpu.SemaphoreType.DMA for _ in range(2)],
        ],
    )
    return gather(tab, zi).reshape(_N, _D)


# speedup vs baseline: 10.4946x; 3.2633x over previous
import jax
import jax.numpy as jnp
from jax import lax
from jax.experimental import pallas as pl
from jax.experimental.pallas import tpu as pltpu
from jax.experimental.pallas import tpu_sc as plsc

_N = 131072
_ZROWS = 86
_D = 256

_NC = 2
_NS = 16
_NW = _NC * _NS
_BPW = _N // _NW
_C = 32
_NCHUNK = _BPW // _C
_NBUF = 8
_ZP = 88
_LOOK = 4


def _prep_body(ec_ref, w_ref, zt_ref, z_ref, tab_ref, idx_ref):
    t = (jnp.dot(ec_ref[...], w_ref[...], preferred_element_type=jnp.float32)
         + zt_ref[...])
    for w in range(_NW):
        tab_ref[pl.ds(w * _ZP, _ZROWS), :] = t
    shift = jax.lax.broadcasted_iota(jnp.int32, (_NW, _BPW), 0) * _ZP
    idx_ref[...] = z_ref[...] + shift


def _prep(ec86, w, zt, z2d):
    return pl.pallas_call(
        _prep_body,
        out_shape=(
            jax.ShapeDtypeStruct((_NW * _ZP, _D), jnp.float32),
            jax.ShapeDtypeStruct((_NW, _BPW), jnp.int32),
        ),
    )(ec86, w, zt, z2d)


def _gather_body(table_hbm, idx_hbm, out_hbm,
                 idx_v, tmp_v, tab_lin, bufs, gsems, osems):
    cid = lax.axis_index("c")
    sid = lax.axis_index("s")
    wid = sid * _NC + cid
    base = wid * _BPW

    pltpu.sync_copy(idx_hbm.at[pl.ds(base, _BPW)], idx_v)
    pltpu.sync_copy(table_hbm.at[pl.ds(wid * _ZP, _ZP)], tmp_v)
    pltpu.sync_copy(tmp_v, tab_lin.at[pl.ds(wid * _ZP, _ZP)])

    def start_gather(gi, b):
        pltpu.async_copy(tab_lin.at[idx_v.at[pl.ds(gi * _C, _C)]],
                         bufs[b], gsems[b])

    def wait_gather(b):
        pltpu.make_async_copy(out_hbm.at[pl.ds(base, _C)],
                              bufs[b], gsems[b]).wait()

    def start_scatter(gi, b):
        pltpu.async_copy(bufs[b], out_hbm.at[pl.ds(base + gi * _C, _C)],
                         osems[b])

    def wait_scatter(b):
        pltpu.make_async_copy(bufs[b], out_hbm.at[pl.ds(base, _C)],
                              osems[b]).wait()

    for k in range(_LOOK):
        start_gather(k, k)

    @pl.loop(0, _NCHUNK, step=_NBUF)
    def _chunks(g):
        for u in range(_NBUF):
            gi = g + u
            b = u
            bn = (u + _LOOK) % _NBUF
            wait_gather(b)
            start_scatter(gi, b)
            nxt = gi + _LOOK

            @pl.when(nxt < _NCHUNK)
            def _():
                @pl.when(nxt >= _NBUF)
                def _():
                    wait_scatter(bn)
                start_gather(nxt, bn)

    for k in range(_NBUF, 0, -1):
        wait_scatter((_NCHUNK - k) % _NBUF)


def kernel(z, elec_config, m_weight, z_table):
    zi = z.astype(jnp.int32).reshape(_NW, _BPW)
    tab, idx = _prep(elec_config[:_ZROWS], m_weight, z_table, zi)
    idx = idx.reshape(_N)
    mesh = plsc.VectorSubcoreMesh(core_axis_name="c", subcore_axis_name="s",
                                  num_cores=_NC, num_subcores=_NS)
    gather = pl.kernel(
        _gather_body,
        out_type=jax.ShapeDtypeStruct((_N, _D), jnp.float32),
        mesh=mesh,
        scratch_types=[
            pltpu.VMEM((_BPW,), jnp.int32),
            pltpu.VMEM((_ZP, _D), jnp.float32),
            pltpu.HBM((_NW * _ZP, _D), jnp.float32),
            [pltpu.VMEM((_C, _D), jnp.float32) for _ in range(_NBUF)],
            [pltpu.SemaphoreType.DMA for _ in range(_NBUF)],
            [pltpu.SemaphoreType.DMA for _ in range(_NBUF)],
        ],
    )
    return gather(tab, idx)


# speedup vs baseline: 10.8773x; 1.0365x over previous
import jax
import jax.numpy as jnp
from jax import lax
from jax.experimental import pallas as pl
from jax.experimental.pallas import tpu as pltpu
from jax.experimental.pallas import tpu_sc as plsc

_N = 131072
_ZROWS = 86
_ZP = 88
_D = 256
_L = 16

_NC = 2
_NS = 16
_NW = _NC * _NS
_BPW = _N // _NW

_CS = 32
_NBUF = 4
_LOOK = 2
_CR = 128
_NRC = 8
_SPLIT = _BPW - _NRC * _CR
_NSC = _SPLIT // _CS
_JJ = _NSC // _NRC
_RN = [11] * 8 + [10] * 4
_ROFF = [sum(_RN[:j]) for j in range(_JJ)]


def _prep_body(ec_ref, w_ref, zt_ref, z_ref, tab_ref, idx_ref):
    t = (jnp.dot(ec_ref[...], w_ref[...], preferred_element_type=jnp.float32)
         + zt_ref[...])
    for w in range(_NW):
        tab_ref[pl.ds(w * _ZP, _ZROWS), :] = t
    shift = jax.lax.broadcasted_iota(jnp.int32, (_NW, _BPW), 0) * _ZP
    idx_ref[...] = z_ref[...] + shift


def _prep(ec86, w, zt, z2d):
    return pl.pallas_call(
        _prep_body,
        out_shape=(
            jax.ShapeDtypeStruct((_NW * _ZP, _D), jnp.float32),
            jax.ShapeDtypeStruct((_NW, _BPW), jnp.int32),
        ),
    )(ec86, w, zt, z2d)


def _gather_body(table_hbm, idx_hbm, out_hbm,
                 idx_v, table_v, sbufs, rbufs, gsems, osems, rsems):
    cid = lax.axis_index("c")
    sid = lax.axis_index("s")
    wid = sid * _NC + cid
    base = wid * _BPW

    pltpu.sync_copy(idx_hbm.at[pl.ds(base, _BPW)], idx_v)
    pltpu.sync_copy(table_hbm.at[pl.ds(wid * _ZP, _ZP)], table_v)

    iota16 = jax.lax.iota(jnp.int32, _L)

    def start_gather(gi, b):
        pltpu.async_copy(table_hbm.at[idx_v.at[pl.ds(gi * _CS, _CS)]],
                         sbufs[b], gsems[b])

    def wait_gather(b):
        pltpu.make_async_copy(out_hbm.at[pl.ds(base, _CS)],
                              sbufs[b], gsems[b]).wait()

    def start_scatter(gi, b):
        pltpu.async_copy(sbufs[b], out_hbm.at[pl.ds(base + gi * _CS, _CS)],
                         osems[b])

    def wait_scatter(b):
        pltpu.make_async_copy(sbufs[b], out_hbm.at[pl.ds(base, _CS)],
                              osems[b]).wait()

    def start_rscatter(rc, p):
        pltpu.async_copy(
            rbufs[p],
            out_hbm.at[pl.ds(base + _SPLIT + rc * _CR, _CR)], rsems[p])

    def wait_rscatter(p):
        pltpu.make_async_copy(rbufs[p], out_hbm.at[pl.ds(base, _CR)],
                              rsems[p]).wait()

    for k in range(_LOOK):
        start_gather(k, k)

    @pl.loop(0, _NRC, step=2)
    def _regchunks(rc0):
        for p in range(2):
            rc = rc0 + p

            @pl.when(rc >= 2)
            def _():
                wait_rscatter(p)

            for j in range(_JJ):
                sci = rc * _JJ + j
                b = j % _NBUF
                bn = (j + _LOOK) % _NBUF
                wait_gather(b)
                start_scatter(sci, b)
                nxt = sci + _LOOK

                @pl.when(nxt < _NSC)
                def _():
                    @pl.when(nxt >= _NBUF)
                    def _():
                        wait_scatter(bn)
                    start_gather(nxt, bn)

                @pl.loop(0, _RN[j])
                def _rows(k):
                    rr = _ROFF[j] + k
                    ridx = _SPLIT + rc * _CR + rr
                    zsp = plsc.load_gather(
                        idx_v, [jnp.full((_L,), ridx, jnp.int32)])
                    zloc = zsp - wid * _ZP
                    for c in range(_D // _L):
                        val = plsc.load_gather(
                            table_v, [zloc, iota16 + _L * c])
                        rbufs[p][rr, pl.ds(_L * c, _L)] = val

            start_rscatter(rc, p)

    for k in range(_NBUF, 0, -1):
        wait_scatter((_NSC - k) % _NBUF)
    wait_rscatter(0)
    wait_rscatter(1)


def kernel(z, elec_config, m_weight, z_table):
    zi = z.astype(jnp.int32).reshape(_NW, _BPW)
    tab, idx = _prep(elec_config[:_ZROWS], m_weight, z_table, zi)
    idx = idx.reshape(_N)
    mesh = plsc.VectorSubcoreMesh(core_axis_name="c", subcore_axis_name="s",
                                  num_cores=_NC, num_subcores=_NS)
    gather = pl.kernel(
        _gather_body,
        out_type=jax.ShapeDtypeStruct((_N, _D), jnp.float32),
        mesh=mesh,
        compiler_params=pltpu.CompilerParams(needs_layout_passes=False),
        scratch_types=[
            pltpu.VMEM((_BPW,), jnp.int32),
            pltpu.VMEM((_ZP, _D), jnp.float32),
            [pltpu.VMEM((_CS, _D), jnp.float32) for _ in range(_NBUF)],
            [pltpu.VMEM((_CR, _D), jnp.float32) for _ in range(2)],
            [pltpu.SemaphoreType.DMA for _ in range(_NBUF)],
            [pltpu.SemaphoreType.DMA for _ in range(_NBUF)],
            [pltpu.SemaphoreType.DMA for _ in range(2)],
        ],
    )
    return gather(tab, idx)


# speedup vs baseline: 10.9433x; 1.0061x over previous
import jax
import jax.numpy as jnp
from jax import lax
from jax.experimental import pallas as pl
from jax.experimental.pallas import tpu as pltpu
from jax.experimental.pallas import tpu_sc as plsc

_N = 131072
_ZROWS = 86
_ZP = 88
_D = 256
_L = 16

_NC = 2
_NS = 16
_NW = _NC * _NS
_BPW = _N // _NW

_CS = 32
_NBUF = 4
_LOOK = 2
_CR = 128
_NRC = 8
_SPLIT = _BPW - _NRC * _CR
_NSC = _SPLIT // _CS
_JJ = _NSC // _NRC
_RN = [11] * 8 + [10] * 4
_ROFF = [sum(_RN[:j]) for j in range(_JJ)]


def _prep_body(ec_ref, w_ref, zt_ref, z_ref, tab_ref, idx_ref):
    t = (jnp.dot(ec_ref[...], w_ref[...], preferred_element_type=jnp.float32)
         + zt_ref[...])
    for w in range(_NW):
        tab_ref[pl.ds(w * _ZP, _ZROWS), :] = t
    shift = jax.lax.broadcasted_iota(jnp.int32, (_NW, _BPW), 0) * _ZP
    idx_ref[...] = z_ref[...] + shift


def _prep(ec86, w, zt, z2d):
    return pl.pallas_call(
        _prep_body,
        out_shape=(
            jax.ShapeDtypeStruct((_NW * _ZP, _D), jnp.float32),
            jax.ShapeDtypeStruct((_NW, _BPW), jnp.int32),
        ),
    )(ec86, w, zt, z2d)


def _gather_body(table_hbm, idx_hbm, out_hbm,
                 idx_v, table_v, sbufs, rbufs, gsems, osems, rsems):
    cid = lax.axis_index("c")
    sid = lax.axis_index("s")
    wid = sid * _NC + cid
    base = wid * _BPW

    pltpu.sync_copy(idx_hbm.at[pl.ds(base, _BPW)], idx_v)
    pltpu.sync_copy(table_hbm.at[pl.ds(wid * _ZP, _ZP)], table_v)

    iota16 = jax.lax.iota(jnp.int32, _L)

    def start_gather(gi, b):
        pltpu.async_copy(table_hbm.at[idx_v.at[pl.ds(gi * _CS, _CS)]],
                         sbufs[b], gsems[b])

    def wait_gather(b):
        pltpu.make_async_copy(out_hbm.at[pl.ds(base, _CS)],
                              sbufs[b], gsems[b]).wait()

    def start_scatter(gi, b):
        pltpu.async_copy(sbufs[b], out_hbm.at[pl.ds(base + gi * _CS, _CS)],
                         osems[b])

    def wait_scatter(b):
        pltpu.make_async_copy(sbufs[b], out_hbm.at[pl.ds(base, _CS)],
                              osems[b]).wait()

    def start_rscatter(rc, p):
        pltpu.async_copy(
            rbufs[p],
            out_hbm.at[pl.ds(base + _SPLIT + rc * _CR, _CR)], rsems[p])

    def wait_rscatter(p):
        pltpu.make_async_copy(rbufs[p], out_hbm.at[pl.ds(base, _CR)],
                              rsems[p]).wait()

    for k in range(_LOOK):
        start_gather(k, k)

    @pl.loop(0, _NRC, step=2)
    def _regchunks(rc0):
        for p in range(2):
            rc = rc0 + p

            @pl.when(rc >= 2)
            def _():
                wait_rscatter(p)

            for j in range(_JJ):
                sci = rc * _JJ + j
                b = j % _NBUF
                bn = (j + _LOOK) % _NBUF
                wait_gather(b)
                start_scatter(sci, b)
                nxt = sci + _LOOK

                @pl.when(nxt < _NSC)
                def _():
                    @pl.when(nxt >= _NBUF)
                    def _():
                        wait_scatter(bn)
                    start_gather(nxt, bn)

                @plsc.parallel_loop(0, _RN[j], 1)
                def _rows(k):
                    rr = _ROFF[j] + k
                    ridx = _SPLIT + rc * _CR + rr
                    zsp = plsc.load_gather(
                        idx_v, [jnp.full((_L,), ridx, jnp.int32)])
                    zloc = zsp - wid * _ZP
                    for c in range(_D // _L):
                        val = plsc.load_gather(
                            table_v, [zloc, iota16 + _L * c])
                        rbufs[p][rr, pl.ds(_L * c, _L)] = val

            start_rscatter(rc, p)

    for k in range(_NBUF, 0, -1):
        wait_scatter((_NSC - k) % _NBUF)
    wait_rscatter(0)
    wait_rscatter(1)


def kernel(z, elec_config, m_weight, z_table):
    zi = z.astype(jnp.int32).reshape(_NW, _BPW)
    tab, idx = _prep(elec_config[:_ZROWS], m_weight, z_table, zi)
    idx = idx.reshape(_N)
    mesh = plsc.VectorSubcoreMesh(core_axis_name="c", subcore_axis_name="s",
                                  num_cores=_NC, num_subcores=_NS)
    gather = pl.kernel(
        _gather_body,
        out_type=jax.ShapeDtypeStruct((_N, _D), jnp.float32),
        mesh=mesh,
        compiler_params=pltpu.CompilerParams(needs_layout_passes=False),
        scratch_types=[
            pltpu.VMEM((_BPW,), jnp.int32),
            pltpu.VMEM((_ZP, _D), jnp.float32),
            [pltpu.VMEM((_CS, _D), jnp.float32) for _ in range(_NBUF)],
            [pltpu.VMEM((_CR, _D), jnp.float32) for _ in range(2)],
            [pltpu.SemaphoreType.DMA for _ in range(_NBUF)],
            [pltpu.SemaphoreType.DMA for _ in range(_NBUF)],
            [pltpu.SemaphoreType.DMA for _ in range(2)],
        ],
    )
    return gather(tab, idx)
